# baseline (speedup 1.0000x reference)
import functools

import jax
import jax.numpy as jnp
import numpy as np
from jax import lax
from jax.experimental import pallas as pl
from jax.experimental.pallas import tpu as pltpu
from jax.experimental.pallas import tpu_sc as plsc

B = 16384
F = 26
V = 100000
D = 16

NC = 2
NS = 16
L = 16
NW = NC * NS
BPW = B // NW
W2R = F * V * D // 128
FB = ((0, 0), (1, F - L))

_DN = lax.GatherDimensionNumbers(
    offset_dims=(), collapsed_slice_dims=(0,), start_index_map=(0,))


def _shuf(x, perm):
    return lax.gather(x, perm[:, None], dimension_numbers=_DN,
                      slice_sizes=(1,),
                      mode=lax.GatherScatterMode.PROMISE_IN_BOUNDS)


def _lane_sum(x, lanes):
    for k in (8, 4, 2, 1):
        x = x + _shuf(x, lanes ^ k)
    return x


def _transpose16(vs, lanes):
    for k in (8, 4, 2, 1):
        vs = [jnp.where((lanes & k) == (i & k), vs[i],
                        _shuf(vs[i ^ k], lanes ^ k))
              for i in range(L)]
    return vs


C2 = 32
NCHUNK2 = BPW // C2
N2 = C2 * F


def _so_body(xi_hbm, xv_hbm, w2_hbm, out_hbm,
             idxr_v, xvr_v, xvt_v, rowl_v, sub_v, rows_v, ob_v, sem):
    wid = lax.axis_index("s") * NC + lax.axis_index("c")
    base = wid * BPW
    lanes = lax.iota(jnp.int32, L)

    def chunk_body(ck, _):
        b0 = pl.multiple_of(base + ck * C2, C2)

        pltpu.sync_copy(xi_hbm.at[pl.ds(b0, C2), :], idxr_v)
        pltpu.sync_copy(xv_hbm.at[pl.ds(b0, C2), :], xvr_v)
        for bg in range(C2 // L):
            for fb, f0 in FB:
                iv = [idxr_v[bg * L + i, pl.ds(f0, L)] for i in range(L)]
                vv = [xvr_v[bg * L + i, pl.ds(f0, L)] for i in range(L)]
                it = _transpose16(iv, lanes)
                vt = _transpose16(vv, lanes)
                for j in (range(L) if fb == 0 else range(2 * L - F, L)):
                    f = f0 + j
                    flat = it[j] + jnp.int32(f * V)
                    rowl_v[f, pl.ds(bg * L, L)] = (
                        lax.shift_right_logical(flat, 3))
                    sub_v[f, pl.ds(bg * L, L)] = (flat & 7) * L
                    xvt_v[f, pl.ds(bg * L, L)] = vt[j]

        copies = []
        for f in range(F):
            copies.append(pltpu.async_copy(
                w2_hbm.at[rowl_v.at[f]],
                rows_v.at[pl.ds(f * C2, C2), :], sem))
        for cp in copies:
            cp.wait()

        for bg in range(C2 // L):
            ovec = jnp.zeros((L,), jnp.float32)
            xvl = []
            subl = []
            for f in range(F):
                xvl.append(xvt_v[f, pl.ds(bg * L, L)])
                subl.append(sub_v[f, pl.ds(bg * L, L)])
            for i in range(L):
                b = bg * L + i
                acc = jnp.zeros((L,), jnp.float32)
                acc2 = jnp.zeros((L,), jnp.float32)
                for f in range(F):
                    t = rows_v[f * C2 + b, pl.ds(subl[f][i], L)] * xvl[f][i]
                    acc = acc + t
                    acc2 = acc2 + t * t
                s = _lane_sum(acc * acc - acc2, lanes)
                ovec = jnp.where(lanes == i, ovec + jnp.float32(0.5) * s, ovec)
            ob_v[pl.ds(ck * C2 + bg * L, L)] = ovec
        return 0

    lax.fori_loop(0, NCHUNK2, chunk_body, 0)
    pltpu.sync_copy(ob_v, out_hbm.at[wid])


C1 = 128
NCHUNK1 = BPW // C1
N1 = C1 * F


def _fo_body(xi_hbm, xv_hbm, w1_hbm, out_hbm,
             idxr_v, idxt_v, xvt_v, xv_v, w1t_v, ob_v, sem):
    wid = lax.axis_index("s") * NC + lax.axis_index("c")
    base = wid * BPW
    lanes = lax.iota(jnp.int32, L)

    def chunk_body(ck, _):
        b0 = pl.multiple_of(base + ck * C1, C1)
        q0 = pl.multiple_of((base + ck * C1) * F, 128)

        pltpu.sync_copy(xi_hbm.at[pl.ds(q0, N1)], idxr_v.at[pl.ds(0, N1)])
        pltpu.sync_copy(xv_hbm.at[pl.ds(q0, N1)], xv_v.at[pl.ds(0, N1)])

        def tbody(bg, _):
            jb = bg * (L * F)
            for fb, f0 in FB:
                iv = [idxr_v[pl.ds(jb + i * F + f0, L)] for i in range(L)]
                vv = [xv_v[pl.ds(jb + i * F + f0, L)] for i in range(L)]
                it = _transpose16(iv, lanes)
                vt = _transpose16(vv, lanes)
                for j in (range(L) if fb == 0 else range(2 * L - F, L)):
                    f = f0 + j
                    idxt_v[f, pl.ds(bg * L, L)] = it[j]
                    xvt_v[f, pl.ds(bg * L, L)] = vt[j]
            return 0

        lax.fori_loop(0, C1 // L, tbody, 0)

        copies = []
        for f in range(F):
            copies.append(pltpu.async_copy(
                w1_hbm.at[f].at[idxt_v.at[f]],
                w1t_v.at[pl.ds(f * C1, C1)], sem))
        for cp in copies:
            cp.wait()

        def bbody(bg, _):
            facc = jnp.zeros((L,), jnp.float32)
            for f in range(F):
                facc = facc + (w1t_v[pl.ds(f * C1 + bg * L, L)]
                               * xvt_v[f, pl.ds(bg * L, L)])
            ob_v[pl.ds(bg * L, L)] = facc
            return 0

        lax.fori_loop(0, C1 // L, bbody, 0)
        pltpu.sync_copy(ob_v, out_hbm.at[pl.ds(b0, C1)])
        return 0

    lax.fori_loop(0, NCHUNK1, chunk_body, 0)


@jax.jit
def _fm_kernel(xi_2d, xi_flat, xv_2d, xv_flat, w1_2d, w2_rows):
    mesh = plsc.VectorSubcoreMesh(core_axis_name="c", subcore_axis_name="s")
    so = pl.kernel(
        _so_body,
        mesh=mesh,
        compiler_params=pltpu.CompilerParams(use_tc_tiling_on_sc=True),
        out_type=jax.ShapeDtypeStruct((NW, BPW), jnp.float32),
        scratch_types=[
            pltpu.VMEM((C2, F), jnp.int32),
            pltpu.VMEM((C2, F), jnp.float32),
            pltpu.VMEM((F, C2), jnp.float32),
            pltpu.VMEM((F, C2), jnp.int32),
            pltpu.VMEM((F, C2), jnp.int32),
            pltpu.VMEM((N2, 128), jnp.float32),
            pltpu.VMEM((BPW,), jnp.float32),
            pltpu.SemaphoreType.DMA,
        ],
    )
    fo = pl.kernel(
        _fo_body,
        mesh=mesh,
        compiler_params=pltpu.CompilerParams(use_tc_tiling_on_sc=False),
        out_type=jax.ShapeDtypeStruct((B,), jnp.float32),
        scratch_types=[
            pltpu.VMEM((N1 + L,), jnp.int32),
            pltpu.VMEM((F, C1), jnp.int32),
            pltpu.VMEM((F, C1), jnp.float32),
            pltpu.VMEM((N1 + L,), jnp.float32),
            pltpu.VMEM((N1,), jnp.float32),
            pltpu.VMEM((C1,), jnp.float32),
            pltpu.SemaphoreType.DMA,
        ],
    )
    return (so(xi_2d, xv_2d, w2_rows).reshape(B)
            + fo(xi_flat, xv_flat, w1_2d))


def kernel(Xi, Xv, W1, W2, bias):
    xi_2d = Xi[:, :, 0].astype(jnp.int32)
    xi_flat = xi_2d.reshape(B * F)
    xv_flat = Xv.reshape(B * F)
    w1_2d = W1.transpose(0, 2, 1).reshape(F, V)
    w2_rows = W2.reshape(W2R, 128)
    return _fm_kernel(xi_2d, xi_flat, Xv, xv_flat, w1_2d, w2_rows) + bias


# speedup vs baseline: 1.1197x; 1.1197x over previous
import functools

import jax
import jax.numpy as jnp
import numpy as np
from jax import lax
from jax.experimental import pallas as pl
from jax.experimental.pallas import tpu as pltpu
from jax.experimental.pallas import tpu_sc as plsc

B = 16384
F = 26
V = 100000
D = 16

NC = 2
NS = 16
L = 16
NW = NC * NS
BPW = B // NW
C = 128
NCHUNK = BPW // C
N = C * F
NFB = (F + L - 1) // L


def _fm_body(xi_hbm, xv_hbm, w1_hbm, w2_hbm, out_hbm,
             idxr_v, idxt_v, xv_v, xvt_v, w1t_v, rows_v, ob_v, sem):
    wid = lax.axis_index("s") * NC + lax.axis_index("c")
    base = wid * BPW
    lanes = lax.iota(jnp.int32, L)
    _dn = lax.GatherDimensionNumbers(
        offset_dims=(), collapsed_slice_dims=(0,), start_index_map=(0,))

    def _shuf(x, perm):
        return lax.gather(x, perm[:, None], dimension_numbers=_dn,
                          slice_sizes=(1,),
                          mode=lax.GatherScatterMode.PROMISE_IN_BOUNDS)

    def _lane_sum(x):
        for k in (8, 4, 2, 1):
            x = x + _shuf(x, lanes ^ k)
        return x

    def _transpose16(vs):
        for k in (8, 4, 2, 1):
            vs = [jnp.where((lanes & k) == (i & k), vs[i],
                            _shuf(vs[i ^ k], lanes ^ k))
                  for i in range(L)]
        return vs

    def chunk_body(ck, _):
        b0 = pl.multiple_of(base + ck * C, 128)
        q0 = pl.multiple_of((base + ck * C) * F, 128)

        pltpu.sync_copy(xi_hbm.at[pl.ds(q0, N)], idxr_v.at[pl.ds(0, N)])
        pltpu.sync_copy(xv_hbm.at[pl.ds(q0, N)], xv_v.at[pl.ds(0, N)])

        def tbody(bg, _):
            jb = bg * (L * F)
            for fb in range(NFB):
                nf = min(L, F - fb * L)
                iv = [idxr_v[pl.ds(jb + i * F + fb * L, L)] for i in range(L)]
                vv = [xv_v[pl.ds(jb + i * F + fb * L, L)] for i in range(L)]
                it = _transpose16(iv)
                vt = _transpose16(vv)
                for j in range(nf):
                    f = fb * L + j
                    idxt_v[f, pl.ds(bg * L, L)] = it[j]
                    xvt_v[f, pl.ds(bg * L, L)] = vt[j]
            return 0

        lax.fori_loop(0, C // L, tbody, 0)

        copies = []
        for f in range(F):
            copies.append(pltpu.async_copy(
                w2_hbm.at[f].at[idxt_v.at[f]],
                rows_v.at[pl.ds(f * C, C), :], sem))
            copies.append(pltpu.async_copy(
                w1_hbm.at[f].at[idxt_v.at[f]],
                w1t_v.at[pl.ds(f * C, C)], sem))
        for cp in copies:
            cp.wait()

        def bbody(bg, _):
            facc = jnp.zeros((L,), jnp.float32)
            xvl = []
            for f in range(F):
                xvrow = xvt_v[f, pl.ds(bg * L, L)]
                w1row = w1t_v[pl.ds(f * C + bg * L, L)]
                facc = facc + w1row * xvrow
                xvl.append(xvrow)
            ovec = facc
            for i in range(L):
                b = bg * L + i
                acc = jnp.zeros((L,), jnp.float32)
                acc2 = jnp.zeros((L,), jnp.float32)
                for f in range(F):
                    t = rows_v[f * C + b, :] * xvl[f][i]
                    acc = acc + t
                    acc2 = acc2 + t * t
                s = _lane_sum(acc * acc - acc2)
                ovec = jnp.where(lanes == i, ovec + jnp.float32(0.5) * s, ovec)
            ob_v[pl.ds(bg * L, L)] = ovec
            return 0

        lax.fori_loop(0, C // L, bbody, 0)
        pltpu.sync_copy(ob_v, out_hbm.at[pl.ds(b0, C)])
        return 0

    lax.fori_loop(0, NCHUNK, chunk_body, 0)


@jax.jit
def _fm_kernel(xi_flat, xv_flat, w1_flat, w2_3d):
    mesh = plsc.VectorSubcoreMesh(core_axis_name="c", subcore_axis_name="s")
    run = pl.kernel(
        _fm_body,
        mesh=mesh,
        compiler_params=pltpu.CompilerParams(use_tc_tiling_on_sc=False),
        out_type=jax.ShapeDtypeStruct((B,), jnp.float32),
        scratch_types=[
            pltpu.VMEM((N + L,), jnp.int32),
            pltpu.VMEM((F, C), jnp.int32),
            pltpu.VMEM((N + L,), jnp.float32),
            pltpu.VMEM((F, C), jnp.float32),
            pltpu.VMEM((N,), jnp.float32),
            pltpu.VMEM((N, D), jnp.float32),
            pltpu.VMEM((C,), jnp.float32),
            pltpu.SemaphoreType.DMA,
        ],
    )
    return run(xi_flat, xv_flat, w1_flat, w2_3d)


def kernel(Xi, Xv, W1, W2, bias):
    xi_flat = Xi.reshape(B * F).astype(jnp.int32)
    xv_flat = Xv.reshape(B * F)
    w1_2d = W1.transpose(0, 2, 1).reshape(F, V)
    return _fm_kernel(xi_flat, xv_flat, w1_2d, W2) + bias
